# Initial kernel scaffold; baseline (speedup 1.0000x reference)
import jax
import jax.numpy as jnp
from jax.experimental import pallas as pl


def kernel(inputs, W0, b0, W1, b1, Wh, bh):
    raise NotImplementedError("write your pallas kernel here")


# speedup vs baseline: 1.8780x; 1.8780x over previous
import functools

import jax
import jax.numpy as jnp
from jax.experimental import pallas as pl

F = 51
FP = 64
K = 128
BINS = 2
T = 2048


def _fused_kernel(prev_ref, cur_ref, next_ref,
                  w0n_ref, w0p_ref, w1n_ref, w1p_ref, w1y_ref,
                  whn_ref, whp_ref, why_ref,
                  b0_ref, b1_ref, bh_ref,
                  out_ref, *, n_rows):
    i = pl.program_id(0)
    f32 = jnp.float32
    xh = jnp.concatenate(
        [prev_ref[T - 3:, :], cur_ref[...], next_ref[:3, :]], axis=0)
    xneg = jnp.minimum(xh, 0.0)
    xpos = jnp.maximum(xh, 0.0)

    acc0 = jnp.broadcast_to(b0_ref[...], (T + 4, K)).astype(f32)
    for t in range(3):
        acc0 = acc0 + jnp.dot(xneg[t:t + T + 4], w0n_ref[t],
                              preferred_element_type=f32)
        acc0 = acc0 + jnp.dot(xpos[t:t + T + 4], w0p_ref[t],
                              preferred_element_type=f32)
    y0 = jnp.maximum(acc0, 0.0)
    gr = i * T - 2 + jax.lax.broadcasted_iota(jnp.int32, (T + 4, 1), 0)
    y0 = jnp.where((gr >= 0) & (gr < n_rows), y0, 0.0)

    acc1 = jnp.broadcast_to(b1_ref[...], (T, K)).astype(f32)
    for t in range(5):
        acc1 = acc1 + jnp.dot(xneg[1 + t:1 + t + T], w1n_ref[t],
                              preferred_element_type=f32)
        acc1 = acc1 + jnp.dot(xpos[1 + t:1 + t + T], w1p_ref[t],
                              preferred_element_type=f32)
        acc1 = acc1 + jnp.dot(y0[t:t + T], w1y_ref[t],
                              preferred_element_type=f32)
    y1 = jnp.maximum(acc1, 0.0)

    z = (jnp.dot(xneg[3:3 + T], whn_ref[...], preferred_element_type=f32)
         + jnp.dot(xpos[3:3 + T], whp_ref[...], preferred_element_type=f32)
         + jnp.dot(y1, why_ref[...], preferred_element_type=f32)
         + bh_ref[...])
    out_ref[...] = jax.nn.softplus(z)


def _pad_cin(w, cin_pad):
    pad = [(0, 0)] * (w.ndim - 2) + [(0, cin_pad - w.shape[-2]), (0, 0)]
    return jnp.pad(w, pad)


@functools.partial(jax.jit, static_argnums=())
def kernel(inputs, W0, b0, W1, b1, Wh, bh):
    x = inputs[0]
    n, f = x.shape
    nb = n // T
    xpad = jnp.zeros((n + 2 * T, FP), x.dtype).at[T:T + n, :f].set(x)

    w0n = _pad_cin(W0[:, :F, :], FP)
    w0p = _pad_cin(W0[:, F:2 * F, :], FP)
    w1n = _pad_cin(W1[:, :F, :], FP)
    w1p = _pad_cin(W1[:, F:2 * F, :], FP)
    w1y = W1[:, 2 * F:, :]
    whn = _pad_cin(Wh[:F, :], FP)
    whp = _pad_cin(Wh[F:2 * F, :], FP)
    why = Wh[2 * F:, :]
    b0r = b0.reshape(1, K)
    b1r = b1.reshape(1, K)
    bhr = bh.reshape(1, BINS)

    full = lambda shape: pl.BlockSpec(shape, lambda i: (0,) * len(shape))
    out = pl.pallas_call(
        functools.partial(_fused_kernel, n_rows=n),
        grid=(nb,),
        in_specs=[
            pl.BlockSpec((T, FP), lambda i: (i, 0)),
            pl.BlockSpec((T, FP), lambda i: (i + 1, 0)),
            pl.BlockSpec((T, FP), lambda i: (i + 2, 0)),
            full((3, FP, K)), full((3, FP, K)),
            full((5, FP, K)), full((5, FP, K)), full((5, K, K)),
            full((FP, BINS)), full((FP, BINS)), full((K, BINS)),
            full((1, K)), full((1, K)), full((1, BINS)),
        ],
        out_specs=pl.BlockSpec((T, BINS), lambda i: (i, 0)),
        out_shape=jax.ShapeDtypeStruct((n, BINS), jnp.float32),
    )(xpad, xpad, xpad, w0n, w0p, w1n, w1p, w1y, whn, whp, why,
      b0r, b1r, bhr)
    return out[None, :, :]
